# Initial kernel scaffold; baseline (speedup 1.0000x reference)
import jax
import jax.numpy as jnp
from jax.experimental import pallas as pl


def kernel(x, edge_index, edge_attr, W_msg, b_msg, W_apply, b_apply, W_pred, b_pred):
    raise NotImplementedError("write your pallas kernel here")


# speedup vs baseline: 3.8530x; 3.8530x over previous
import functools

import jax
import jax.numpy as jnp
from jax import lax
from jax.experimental import pallas as pl
from jax.experimental.pallas import tpu as pltpu
from jax.experimental.pallas import tpu_sc as plsc

N = 10000
E = 320000
D_IN = 128
D_E = 16
D_OUT = 128

NC = 2
NS = 16
EPT = E // NS
K = 80
NCHUNK = EPT // K
RPT = 624
TAIL = N - NS * RPT
ZR = 104

_mesh = plsc.VectorSubcoreMesh(core_axis_name="c", subcore_axis_name="s")


@functools.partial(
    pl.kernel,
    out_type=(
        jax.ShapeDtypeStruct((N, D_IN), jnp.float32),
        jax.ShapeDtypeStruct((N, D_IN), jnp.float32),
    ),
    mesh=_mesh,
    scratch_types=(
        pltpu.VMEM((K,), jnp.int32),
        pltpu.VMEM((K,), jnp.int32),
        pltpu.VMEM((K, D_IN), jnp.float32),
        pltpu.VMEM((K * D_E,), jnp.float32),
        pltpu.VMEM((ZR, D_IN), jnp.float32),
        pltpu.VMEM_SHARED((N, D_IN), jnp.float32),
        pltpu.SemaphoreType.DMA,
    ),
    compiler_params=pltpu.CompilerParams(needs_layout_passes=False),
)
def _segment_kernel(src_hbm, dst_hbm, x_hbm, ea_hbm,
                    segx_out, eac_out,
                    srcv, dstv, rows, eav, z128, acc, sem):
    c = lax.axis_index("c")
    s = lax.axis_index("s")

    zf = jnp.zeros((16,), jnp.float32)

    def z128_body(i, _):
        for j in range(D_IN // 16):
            z128[i, pl.ds(j * 16, 16)] = zf
        return 0
    lax.fori_loop(0, ZR, z128_body, 0)

    r0 = s * RPT
    for step in range(RPT // ZR):
        pltpu.sync_copy(z128, acc.at[pl.ds(r0 + step * ZR, ZR)])

    @pl.when(s == NS - 1)
    def _():
        pltpu.sync_copy(z128.at[pl.ds(0, TAIL)],
                        acc.at[pl.ds(NS * RPT, TAIL)])

    @pl.when(c == 1)
    def _():
        onehot = jnp.where(lax.iota(jnp.int32, 16) == 0, 1.0, 0.0)

        def pre_body(i, _):
            rows[i, pl.ds(D_E, 16)] = onehot
            for j in range(2, D_IN // 16):
                rows[i, pl.ds(j * 16, 16)] = zf
            return 0
        lax.fori_loop(0, K, pre_body, 0)

    plsc.subcore_barrier()

    e0 = s * EPT

    @pl.when(c == 0)
    def _():
        def chunk_body(ci, _):
            base = e0 + ci * K
            pltpu.sync_copy(src_hbm.at[pl.ds(base, K)], srcv)
            pltpu.sync_copy(dst_hbm.at[pl.ds(base, K)], dstv)
            pltpu.async_copy(x_hbm.at[srcv], rows, sem).wait()
            pltpu.sync_copy(rows, acc.at[dstv], add=True)
            return 0
        lax.fori_loop(0, NCHUNK, chunk_body, 0)

    @pl.when(c == 1)
    def _():
        def chunk_body(ci, _):
            base = e0 + ci * K
            pltpu.sync_copy(dst_hbm.at[pl.ds(base, K)], dstv)
            pltpu.sync_copy(ea_hbm.at[pl.ds(base * D_E, K * D_E)], eav)

            def fill_body(r, _):
                rows[r, pl.ds(0, D_E)] = eav[pl.ds(r * D_E, D_E)]
                return 0
            lax.fori_loop(0, K, fill_body, 0)
            pltpu.sync_copy(rows, acc.at[dstv], add=True)
            return 0
        lax.fori_loop(0, NCHUNK, chunk_body, 0)

    plsc.subcore_barrier()

    @pl.when(c == 0)
    def _():
        pltpu.sync_copy(acc.at[pl.ds(r0, RPT)], segx_out.at[pl.ds(r0, RPT)])

        @pl.when(s == NS - 1)
        def _():
            pltpu.sync_copy(acc.at[pl.ds(NS * RPT, TAIL)],
                            segx_out.at[pl.ds(NS * RPT, TAIL)])

    @pl.when(c == 1)
    def _():
        pltpu.sync_copy(acc.at[pl.ds(r0, RPT)], eac_out.at[pl.ds(r0, RPT)])

        @pl.when(s == NS - 1)
        def _():
            pltpu.sync_copy(acc.at[pl.ds(NS * RPT, TAIL)],
                            eac_out.at[pl.ds(NS * RPT, TAIL)])


EPW = E // (NC * NS)


@functools.partial(
    pl.kernel,
    out_type=jax.ShapeDtypeStruct((2 * E,), jnp.float32),
    mesh=_mesh,
    scratch_types=(
        pltpu.VMEM((4 * N,), jnp.float32),
        pltpu.VMEM((EPW,), jnp.int32),
        pltpu.VMEM((EPW,), jnp.int32),
        pltpu.VMEM((2 * EPW,), jnp.float32),
    ),
    compiler_params=pltpu.CompilerParams(needs_layout_passes=False),
)
def _score_kernel(uv_hbm, src_hbm, dst_hbm, score_out, uvv, srcv, dstv, sv):
    c = lax.axis_index("c")
    s = lax.axis_index("s")
    e0 = (c * NS + s) * EPW

    pltpu.sync_copy(uv_hbm, uvv)
    pltpu.sync_copy(src_hbm.at[pl.ds(e0, EPW)], srcv)
    pltpu.sync_copy(dst_hbm.at[pl.ds(e0, EPW)], dstv)

    iota = lax.iota(jnp.int32, 16)

    def body(i, _):
        s16 = srcv[pl.ds(i * 16, 16)] * 4
        d16 = dstv[pl.ds(i * 16, 16)] * 4
        row = (iota + i * 16) * 2
        for col in range(2):
            a = plsc.load_gather(uvv, [s16 + col])
            b = plsc.load_gather(uvv, [d16 + (col + 2)])
            plsc.store_scatter(sv, [row + col], a + b)
        return 0
    lax.fori_loop(0, EPW // 16, body, 0)

    pltpu.sync_copy(sv, score_out.at[pl.ds(2 * e0, 2 * EPW)])


_BN = 1000


def _dense_body(x_ref, sx_ref, eac_ref,
                wx_ref, we_ref, bm_ref, wa1_ref, wa2_ref, ba_ref,
                wuv_ref, buv_ref, uv_ref):
    segx = sx_ref[...]
    sege = eac_ref[:, 0:D_E]
    cnt = eac_ref[:, D_E:D_E + 1]
    summed = (jnp.dot(segx, wx_ref[...], preferred_element_type=jnp.float32)
              + jnp.dot(sege, we_ref[...], preferred_element_type=jnp.float32)
              + cnt * bm_ref[...])
    aggr = summed / jnp.maximum(cnt, 1.0)
    h = jnp.maximum(
        jnp.dot(x_ref[...], wa1_ref[...], preferred_element_type=jnp.float32)
        + jnp.dot(aggr, wa2_ref[...], preferred_element_type=jnp.float32)
        + ba_ref[...], 0.0)
    uv_ref[...] = jnp.dot(h, wuv_ref[...],
                          preferred_element_type=jnp.float32) + buv_ref[...]


_dense = pl.pallas_call(
    _dense_body,
    grid=(N // _BN,),
    in_specs=[
        pl.BlockSpec((_BN, D_IN), lambda i: (i, 0)),
        pl.BlockSpec((_BN, D_IN), lambda i: (i, 0)),
        pl.BlockSpec((_BN, D_IN), lambda i: (i, 0)),
        pl.BlockSpec((D_IN, D_OUT), lambda i: (0, 0)),
        pl.BlockSpec((D_E, D_OUT), lambda i: (0, 0)),
        pl.BlockSpec((1, D_OUT), lambda i: (0, 0)),
        pl.BlockSpec((D_IN, D_OUT), lambda i: (0, 0)),
        pl.BlockSpec((D_OUT, D_OUT), lambda i: (0, 0)),
        pl.BlockSpec((1, D_OUT), lambda i: (0, 0)),
        pl.BlockSpec((D_OUT, 4), lambda i: (0, 0)),
        pl.BlockSpec((1, 4), lambda i: (0, 0)),
    ],
    out_specs=pl.BlockSpec((_BN, 4), lambda i: (i, 0)),
    out_shape=jax.ShapeDtypeStruct((N, 4), jnp.float32),
)


def kernel(x, edge_index, edge_attr, W_msg, b_msg, W_apply, b_apply,
           W_pred, b_pred):
    src = edge_index[0]
    dst = edge_index[1]

    segx, eac = _segment_kernel(src, dst, x, edge_attr.reshape(-1))

    wx = W_msg[:, :D_IN].T
    we = W_msg[:, D_IN:].T
    wa1 = W_apply[:, :D_IN].T
    wa2 = W_apply[:, D_IN:].T
    wuv = jnp.concatenate([W_pred[:, :D_OUT].T, W_pred[:, D_OUT:].T], axis=1)
    buv = jnp.concatenate([b_pred, jnp.zeros((2,), jnp.float32)])[None, :]

    uv = _dense(x, segx, eac,
                wx, we, b_msg[None, :], wa1, wa2, b_apply[None, :],
                wuv, buv)

    return _score_kernel(uv.reshape(-1), src, dst).reshape(E, 2)


# speedup vs baseline: 5.2834x; 1.3713x over previous
import functools

import jax
import jax.numpy as jnp
from jax import lax
from jax.experimental import pallas as pl
from jax.experimental.pallas import tpu as pltpu
from jax.experimental.pallas import tpu_sc as plsc

N = 10000
E = 320000
D_IN = 128
D_E = 16
D_OUT = 128

NC = 2
NS = 16
K = 128
NCHUNK = 156
ETAIL0 = NS * NCHUNK * K
RPT = 624
TAIL = N - NS * RPT
ZR = 104

_mesh = plsc.VectorSubcoreMesh(core_axis_name="c", subcore_axis_name="s")


@functools.partial(
    pl.kernel,
    out_type=(
        jax.ShapeDtypeStruct((N, D_IN), jnp.float32),
        jax.ShapeDtypeStruct((N, D_IN), jnp.float32),
    ),
    mesh=_mesh,
    scratch_types=(
        pltpu.VMEM((2, K), jnp.int32),
        pltpu.VMEM((K,), jnp.int32),
        pltpu.VMEM((K, D_IN), jnp.float32),
        pltpu.VMEM((K, D_E), jnp.float32),
        pltpu.VMEM((ZR, D_IN), jnp.float32),
        pltpu.VMEM_SHARED((N, D_IN), jnp.float32),
        pltpu.SemaphoreType.DMA,
    ),
    compiler_params=pltpu.CompilerParams(needs_layout_passes=False),
)
def _segment_kernel(ei_hbm, x_hbm, ea_hbm,
                    segx_out, eac_out,
                    eiv, dstv, rows, eav, z128, acc, sem):
    c = lax.axis_index("c")
    s = lax.axis_index("s")

    zf = jnp.zeros((16,), jnp.float32)

    def z128_body(i, _):
        for j in range(D_IN // 16):
            z128[i, pl.ds(j * 16, 16)] = zf
        return 0
    lax.fori_loop(0, ZR, z128_body, 0)

    r0 = s * RPT
    for step in range(RPT // ZR):
        pltpu.sync_copy(z128, acc.at[pl.ds(r0 + step * ZR, ZR)])

    @pl.when(s == NS - 1)
    def _():
        pltpu.sync_copy(z128.at[pl.ds(0, TAIL)],
                        acc.at[pl.ds(NS * RPT, TAIL)])

    @pl.when(c == 1)
    def _():
        onehot = jnp.where(lax.iota(jnp.int32, 16) == 0, 1.0, 0.0)

        def pre_body(i, _):
            rows[i, pl.ds(D_E, 16)] = onehot
            for j in range(2, D_IN // 16):
                rows[i, pl.ds(j * 16, 16)] = zf
            return 0
        lax.fori_loop(0, K, pre_body, 0)

    plsc.subcore_barrier()

    e0 = s * (NCHUNK * K)

    def load_dst(base):
        pltpu.sync_copy(ei_hbm.at[:, pl.ds(base, K)], eiv)
        for j in range(K // 16):
            dstv[pl.ds(j * 16, 16)] = eiv[1, pl.ds(j * 16, 16)]

    def chunk0(base):
        load_dst(base)
        pltpu.async_copy(x_hbm.at[eiv.at[0]], rows, sem).wait()
        pltpu.sync_copy(rows, acc.at[dstv], add=True)

    def chunk1(base):
        load_dst(base)
        pltpu.sync_copy(ea_hbm.at[pl.ds(base, K)], eav)

        def fill_body(r, _):
            rows[r, pl.ds(0, D_E)] = eav[r, pl.ds(0, D_E)]
            return 0
        lax.fori_loop(0, K, fill_body, 0)
        pltpu.sync_copy(rows, acc.at[dstv], add=True)

    @pl.when(c == 0)
    def _():
        def chunk_body(ci, _):
            chunk0(e0 + ci * K)
            return 0
        lax.fori_loop(0, NCHUNK, chunk_body, 0)

        @pl.when(s < (E - ETAIL0) // K)
        def _():
            chunk0(ETAIL0 + s * K)

    @pl.when(c == 1)
    def _():
        def chunk_body(ci, _):
            chunk1(e0 + ci * K)
            return 0
        lax.fori_loop(0, NCHUNK, chunk_body, 0)

        @pl.when(s < (E - ETAIL0) // K)
        def _():
            chunk1(ETAIL0 + s * K)

    plsc.subcore_barrier()

    @pl.when(c == 0)
    def _():
        pltpu.sync_copy(acc.at[pl.ds(r0, RPT)], segx_out.at[pl.ds(r0, RPT)])

        @pl.when(s == NS - 1)
        def _():
            pltpu.sync_copy(acc.at[pl.ds(NS * RPT, TAIL)],
                            segx_out.at[pl.ds(NS * RPT, TAIL)])

    @pl.when(c == 1)
    def _():
        pltpu.sync_copy(acc.at[pl.ds(r0, RPT)], eac_out.at[pl.ds(r0, RPT)])

        @pl.when(s == NS - 1)
        def _():
            pltpu.sync_copy(acc.at[pl.ds(NS * RPT, TAIL)],
                            eac_out.at[pl.ds(NS * RPT, TAIL)])


NW = NC * NS
EPW = (E // (NW * K)) * K
STAIL0 = NW * EPW


@functools.partial(
    pl.kernel,
    out_type=jax.ShapeDtypeStruct((E, 2), jnp.float32),
    mesh=_mesh,
    scratch_types=(
        pltpu.VMEM((4 * N,), jnp.float32),
        pltpu.VMEM((2, EPW), jnp.int32),
        pltpu.VMEM((K, 2), jnp.float32),
        pltpu.VMEM((2, K), jnp.int32),
    ),
    compiler_params=pltpu.CompilerParams(needs_layout_passes=False),
)
def _score_kernel(uv_hbm, ei_hbm, score_out, uvv, eiv, svt, eit):
    c = lax.axis_index("c")
    s = lax.axis_index("s")
    w = c * NS + s
    e0 = w * EPW

    pltpu.sync_copy(uv_hbm, uvv)
    pltpu.sync_copy(ei_hbm.at[:, pl.ds(e0, EPW)], eiv)

    iota = lax.iota(jnp.int32, 16)

    def chunk(idx_ref, ci, out_base):
        def body(i, _):
            s16 = idx_ref[0, pl.ds(ci * K + i * 16, 16)] * 4
            d16 = idx_ref[1, pl.ds(ci * K + i * 16, 16)] * 4
            row = iota + i * 16
            for col in range(2):
                a = plsc.load_gather(uvv, [s16 + col])
                b = plsc.load_gather(uvv, [d16 + (col + 2)])
                plsc.store_scatter(svt,
                                   [row, jnp.full((16,), col, jnp.int32)],
                                   a + b)
            return 0
        lax.fori_loop(0, K // 16, body, 0)
        pltpu.sync_copy(svt, score_out.at[pl.ds(out_base, K)])

    def main_body(ci, _):
        chunk(eiv, ci, e0 + ci * K)
        return 0
    lax.fori_loop(0, EPW // K, main_body, 0)

    @pl.when(w < (E - STAIL0) // K)
    def _():
        t0 = STAIL0 + w * K
        pltpu.sync_copy(ei_hbm.at[:, pl.ds(t0, K)], eit)
        chunk(eit, 0, t0)


_BN = 1000


def _dense_body(x_ref, sx_ref, eac_ref,
                wx_ref, we_ref, bm_ref, wa1_ref, wa2_ref, ba_ref,
                wuv_ref, buv_ref, uv_ref):
    segx = sx_ref[...]
    sege = eac_ref[:, 0:D_E]
    cnt = eac_ref[:, D_E:D_E + 1]
    summed = (jnp.dot(segx, wx_ref[...], preferred_element_type=jnp.float32)
              + jnp.dot(sege, we_ref[...], preferred_element_type=jnp.float32)
              + cnt * bm_ref[...])
    aggr = summed / jnp.maximum(cnt, 1.0)
    h = jnp.maximum(
        jnp.dot(x_ref[...], wa1_ref[...], preferred_element_type=jnp.float32)
        + jnp.dot(aggr, wa2_ref[...], preferred_element_type=jnp.float32)
        + ba_ref[...], 0.0)
    uv_ref[...] = jnp.dot(h, wuv_ref[...],
                          preferred_element_type=jnp.float32) + buv_ref[...]


_dense = pl.pallas_call(
    _dense_body,
    grid=(N // _BN,),
    in_specs=[
        pl.BlockSpec((_BN, D_IN), lambda i: (i, 0)),
        pl.BlockSpec((_BN, D_IN), lambda i: (i, 0)),
        pl.BlockSpec((_BN, D_IN), lambda i: (i, 0)),
        pl.BlockSpec((D_IN, D_OUT), lambda i: (0, 0)),
        pl.BlockSpec((D_E, D_OUT), lambda i: (0, 0)),
        pl.BlockSpec((1, D_OUT), lambda i: (0, 0)),
        pl.BlockSpec((D_IN, D_OUT), lambda i: (0, 0)),
        pl.BlockSpec((D_OUT, D_OUT), lambda i: (0, 0)),
        pl.BlockSpec((1, D_OUT), lambda i: (0, 0)),
        pl.BlockSpec((D_OUT, 4), lambda i: (0, 0)),
        pl.BlockSpec((1, 4), lambda i: (0, 0)),
    ],
    out_specs=pl.BlockSpec((_BN, 4), lambda i: (i, 0)),
    out_shape=jax.ShapeDtypeStruct((N, 4), jnp.float32),
)


def kernel(x, edge_index, edge_attr, W_msg, b_msg, W_apply, b_apply,
           W_pred, b_pred):
    segx, eac = _segment_kernel(edge_index, x, edge_attr)

    wx = W_msg[:, :D_IN].T
    we = W_msg[:, D_IN:].T
    wa1 = W_apply[:, :D_IN].T
    wa2 = W_apply[:, D_IN:].T
    wuv = jnp.concatenate([W_pred[:, :D_OUT].T, W_pred[:, D_OUT:].T], axis=1)
    buv = jnp.concatenate([b_pred, jnp.zeros((2,), jnp.float32)])[None, :]

    uv = _dense(x, segx, eac,
                wx, we, b_msg[None, :], wa1, wa2, b_apply[None, :],
                wuv, buv)

    return _score_kernel(uv.reshape(-1), edge_index)


# speedup vs baseline: 6.7122x; 1.2704x over previous
import functools

import jax
import jax.numpy as jnp
from jax import lax
from jax.experimental import pallas as pl
from jax.experimental.pallas import tpu as pltpu
from jax.experimental.pallas import tpu_sc as plsc

N = 10000
E = 320000
D_IN = 128
D_E = 16
D_OUT = 128

NC = 2
NS = 16
K = 128
NCHUNK = 156
ETAIL0 = NS * NCHUNK * K
RPT = 624
TAIL = N - NS * RPT
ZR = 104

_mesh = plsc.VectorSubcoreMesh(core_axis_name="c", subcore_axis_name="s")


@functools.partial(
    pl.kernel,
    out_type=(
        jax.ShapeDtypeStruct((N, D_IN), jnp.float32),
        jax.ShapeDtypeStruct((N, D_IN), jnp.float32),
    ),
    mesh=_mesh,
    scratch_types=(
        pltpu.VMEM((2, K), jnp.int32),
        pltpu.VMEM((K,), jnp.int32),
        pltpu.VMEM((K, D_IN), jnp.float32),
        pltpu.VMEM((D_E, K), jnp.float32),
        pltpu.VMEM((ZR, D_IN), jnp.float32),
        pltpu.VMEM_SHARED((N, D_IN), jnp.float32),
        pltpu.SemaphoreType.DMA,
    ),
    compiler_params=pltpu.CompilerParams(needs_layout_passes=False),
)
def _segment_kernel(ei_hbm, x_hbm, ea_hbm,
                    segx_out, eac_out,
                    eiv, dstv, rows, eav, z128, acc, sem):
    c = lax.axis_index("c")
    s = lax.axis_index("s")

    zf = jnp.zeros((16,), jnp.float32)

    def z128_body(i, _):
        for j in range(D_IN // 16):
            z128[i, pl.ds(j * 16, 16)] = zf
        return 0
    lax.fori_loop(0, ZR, z128_body, 0)

    r0 = s * RPT
    for step in range(RPT // ZR):
        pltpu.sync_copy(z128, acc.at[pl.ds(r0 + step * ZR, ZR)])

    @pl.when(s == NS - 1)
    def _():
        pltpu.sync_copy(z128.at[pl.ds(0, TAIL)],
                        acc.at[pl.ds(NS * RPT, TAIL)])

    @pl.when(c == 1)
    def _():
        onehot = jnp.where(lax.iota(jnp.int32, 16) == 0, 1.0, 0.0)

        def pre_body(i, _):
            rows[i, pl.ds(D_E, 16)] = onehot
            for j in range(2, D_IN // 16):
                rows[i, pl.ds(j * 16, 16)] = zf
            return 0
        lax.fori_loop(0, K, pre_body, 0)

    plsc.subcore_barrier()

    e0 = s * (NCHUNK * K)

    def load_dst(base):
        pltpu.sync_copy(ei_hbm.at[:, pl.ds(base, K)], eiv)
        for j in range(K // 16):
            dstv[pl.ds(j * 16, 16)] = eiv[1, pl.ds(j * 16, 16)]

    def chunk0(base):
        load_dst(base)
        pltpu.async_copy(x_hbm.at[eiv.at[0]], rows, sem).wait()
        pltpu.sync_copy(rows, acc.at[dstv], add=True)

    iota16 = lax.iota(jnp.int32, 16)

    def chunk1(base):
        load_dst(base)
        pltpu.sync_copy(ea_hbm.at[:, pl.ds(base, K)], eav)

        def fill_body(r, _):
            col = plsc.load_gather(eav, [iota16, jnp.full((16,), r, jnp.int32)])
            rows[r, pl.ds(0, D_E)] = col
            return 0
        lax.fori_loop(0, K, fill_body, 0)
        pltpu.sync_copy(rows, acc.at[dstv], add=True)

    @pl.when(c == 0)
    def _():
        def chunk_body(ci, _):
            chunk0(e0 + ci * K)
            return 0
        lax.fori_loop(0, NCHUNK, chunk_body, 0)

        @pl.when(s < (E - ETAIL0) // K)
        def _():
            chunk0(ETAIL0 + s * K)

    @pl.when(c == 1)
    def _():
        def chunk_body(ci, _):
            chunk1(e0 + ci * K)
            return 0
        lax.fori_loop(0, NCHUNK, chunk_body, 0)

        @pl.when(s < (E - ETAIL0) // K)
        def _():
            chunk1(ETAIL0 + s * K)

    plsc.subcore_barrier()

    @pl.when(c == 0)
    def _():
        pltpu.sync_copy(acc.at[pl.ds(r0, RPT)], segx_out.at[pl.ds(r0, RPT)])

        @pl.when(s == NS - 1)
        def _():
            pltpu.sync_copy(acc.at[pl.ds(NS * RPT, TAIL)],
                            segx_out.at[pl.ds(NS * RPT, TAIL)])

    @pl.when(c == 1)
    def _():
        pltpu.sync_copy(acc.at[pl.ds(r0, RPT)], eac_out.at[pl.ds(r0, RPT)])

        @pl.when(s == NS - 1)
        def _():
            pltpu.sync_copy(acc.at[pl.ds(NS * RPT, TAIL)],
                            eac_out.at[pl.ds(NS * RPT, TAIL)])


NW = NC * NS
EPW = (E // (NW * K)) * K
STAIL0 = NW * EPW


@functools.partial(
    pl.kernel,
    out_type=jax.ShapeDtypeStruct((2, E), jnp.float32),
    mesh=_mesh,
    scratch_types=(
        pltpu.VMEM((4 * N,), jnp.float32),
        pltpu.VMEM((2, EPW), jnp.int32),
        pltpu.VMEM((2, EPW), jnp.float32),
        pltpu.VMEM((2, K), jnp.int32),
        pltpu.VMEM((2, K), jnp.float32),
    ),
    compiler_params=pltpu.CompilerParams(needs_layout_passes=False),
)
def _score_kernel(uv_hbm, ei_hbm, score_out, uvv, eiv, sv, eit, svt):
    c = lax.axis_index("c")
    s = lax.axis_index("s")
    w = c * NS + s
    e0 = w * EPW

    pltpu.sync_copy(uv_hbm, uvv)
    pltpu.sync_copy(ei_hbm.at[:, pl.ds(e0, EPW)], eiv)

    def make_body(idx_ref, out_ref):
        def body(i, _):
            s16 = idx_ref[0, pl.ds(i * 16, 16)] * 4
            d16 = idx_ref[1, pl.ds(i * 16, 16)] * 4
            for col in range(2):
                a = plsc.load_gather(uvv, [s16 + col])
                b = plsc.load_gather(uvv, [d16 + (col + 2)])
                out_ref[col, pl.ds(i * 16, 16)] = a + b
            return 0
        return body

    lax.fori_loop(0, EPW // 16, make_body(eiv, sv), 0)
    pltpu.sync_copy(sv, score_out.at[:, pl.ds(e0, EPW)])

    @pl.when(w < (E - STAIL0) // K)
    def _():
        t0 = STAIL0 + w * K
        pltpu.sync_copy(ei_hbm.at[:, pl.ds(t0, K)], eit)
        lax.fori_loop(0, K // 16, make_body(eit, svt), 0)
        pltpu.sync_copy(svt, score_out.at[:, pl.ds(t0, K)])


_BN = 1000


def _dense_body(x_ref, sx_ref, eac_ref,
                wx_ref, we_ref, bm_ref, wa1_ref, wa2_ref, ba_ref,
                wuv_ref, buv_ref, uv_ref):
    segx = sx_ref[...]
    sege = eac_ref[:, 0:D_E]
    cnt = eac_ref[:, D_E:D_E + 1]
    summed = (jnp.dot(segx, wx_ref[...], preferred_element_type=jnp.float32)
              + jnp.dot(sege, we_ref[...], preferred_element_type=jnp.float32)
              + cnt * bm_ref[...])
    aggr = summed / jnp.maximum(cnt, 1.0)
    h = jnp.maximum(
        jnp.dot(x_ref[...], wa1_ref[...], preferred_element_type=jnp.float32)
        + jnp.dot(aggr, wa2_ref[...], preferred_element_type=jnp.float32)
        + ba_ref[...], 0.0)
    uv_ref[...] = jnp.dot(h, wuv_ref[...],
                          preferred_element_type=jnp.float32) + buv_ref[...]


_dense = pl.pallas_call(
    _dense_body,
    grid=(N // _BN,),
    in_specs=[
        pl.BlockSpec((_BN, D_IN), lambda i: (i, 0)),
        pl.BlockSpec((_BN, D_IN), lambda i: (i, 0)),
        pl.BlockSpec((_BN, D_IN), lambda i: (i, 0)),
        pl.BlockSpec((D_IN, D_OUT), lambda i: (0, 0)),
        pl.BlockSpec((D_E, D_OUT), lambda i: (0, 0)),
        pl.BlockSpec((1, D_OUT), lambda i: (0, 0)),
        pl.BlockSpec((D_IN, D_OUT), lambda i: (0, 0)),
        pl.BlockSpec((D_OUT, D_OUT), lambda i: (0, 0)),
        pl.BlockSpec((1, D_OUT), lambda i: (0, 0)),
        pl.BlockSpec((D_OUT, 4), lambda i: (0, 0)),
        pl.BlockSpec((1, 4), lambda i: (0, 0)),
    ],
    out_specs=pl.BlockSpec((_BN, 4), lambda i: (i, 0)),
    out_shape=jax.ShapeDtypeStruct((N, 4), jnp.float32),
)


def kernel(x, edge_index, edge_attr, W_msg, b_msg, W_apply, b_apply,
           W_pred, b_pred):
    segx, eac = _segment_kernel(edge_index, x, edge_attr.T)

    wx = W_msg[:, :D_IN].T
    we = W_msg[:, D_IN:].T
    wa1 = W_apply[:, :D_IN].T
    wa2 = W_apply[:, D_IN:].T
    wuv = jnp.concatenate([W_pred[:, :D_OUT].T, W_pred[:, D_OUT:].T], axis=1)
    buv = jnp.concatenate([b_pred, jnp.zeros((2,), jnp.float32)])[None, :]

    uv = _dense(x, segx, eac,
                wx, we, b_msg[None, :], wa1, wa2, b_apply[None, :],
                wuv, buv)

    return _score_kernel(uv.reshape(-1), edge_index).T


# speedup vs baseline: 9.6125x; 1.4321x over previous
import functools

import jax
import jax.numpy as jnp
from jax import lax
from jax.experimental import pallas as pl
from jax.experimental.pallas import tpu as pltpu
from jax.experimental.pallas import tpu_sc as plsc

N = 10000
E = 320000
D_IN = 128
D_E = 16
D_OUT = 128

NC = 2
NS = 16
K = 128
NCHUNK = 156
ETAIL0 = NS * NCHUNK * K
RPT = 624
TAIL = N - NS * RPT
ZR = 104

_mesh = plsc.VectorSubcoreMesh(core_axis_name="c", subcore_axis_name="s")


@functools.partial(
    pl.kernel,
    out_type=(
        jax.ShapeDtypeStruct((N, D_IN), jnp.float32),
        jax.ShapeDtypeStruct((N, D_IN), jnp.float32),
    ),
    mesh=_mesh,
    scratch_types=(
        pltpu.VMEM((2, K), jnp.int32),
        pltpu.VMEM((2, K), jnp.int32),
        pltpu.VMEM((K,), jnp.int32),
        pltpu.VMEM((K,), jnp.int32),
        pltpu.VMEM((K, D_IN), jnp.float32),
        pltpu.VMEM((K, D_IN), jnp.float32),
        pltpu.VMEM((D_E, K), jnp.float32),
        pltpu.VMEM((D_E, K), jnp.float32),
        pltpu.SemaphoreType.DMA,
        pltpu.SemaphoreType.DMA,
        pltpu.SemaphoreType.DMA,
        pltpu.SemaphoreType.DMA,
        pltpu.SemaphoreType.DMA,
        pltpu.SemaphoreType.DMA,
        pltpu.VMEM_SHARED((N, D_IN), jnp.float32),
    ),
    compiler_params=pltpu.CompilerParams(needs_layout_passes=False),
)
def _segment_kernel(ei_hbm, x_hbm, ea_hbm,
                    segx_out, eac_out,
                    eiv0, eiv1, dst0, dst1, rows0, rows1, eav0, eav1,
                    si0, si1, sg0, sg1, ss0, ss1, acc):
    c = lax.axis_index("c")
    s = lax.axis_index("s")
    EIV = (eiv0, eiv1)
    DST = (dst0, dst1)
    ROWS = (rows0, rows1)
    EAV = (eav0, eav1)
    SI = (si0, si1)
    SG = (sg0, sg1)
    SS = (ss0, ss1)

    zf = jnp.zeros((16,), jnp.float32)

    def zero_body(i, _):
        for j in range(D_IN // 16):
            rows0[i, pl.ds(j * 16, 16)] = zf
            rows1[i, pl.ds(j * 16, 16)] = zf
        return 0
    lax.fori_loop(0, K, zero_body, 0)

    r0 = s * RPT
    for step in range(4):
        pltpu.sync_copy(rows0, acc.at[pl.ds(r0 + step * K, K)])
    pltpu.sync_copy(rows1.at[pl.ds(0, RPT - 4 * K)],
                    acc.at[pl.ds(r0 + 4 * K, RPT - 4 * K)])

    @pl.when(s == NS - 1)
    def _():
        pltpu.sync_copy(rows0.at[pl.ds(0, TAIL)],
                        acc.at[pl.ds(NS * RPT, TAIL)])

    @pl.when(c == 1)
    def _():
        onehot = jnp.where(lax.iota(jnp.int32, 16) == 0, 1.0, 0.0)

        def pre_body(i, _):
            rows0[i, pl.ds(D_E, 16)] = onehot
            rows1[i, pl.ds(D_E, 16)] = onehot
            return 0
        lax.fori_loop(0, K, pre_body, 0)

    plsc.subcore_barrier()

    e0 = s * (NCHUNK * K)
    iota16 = lax.iota(jnp.int32, 16)

    def start_idx(b, base):
        pltpu.async_copy(ei_hbm.at[:, pl.ds(base, K)], EIV[b], SI[b])

    def wait_idx(b):
        pltpu.make_async_copy(ei_hbm.at[:, pl.ds(0, K)], EIV[b], SI[b]).wait()

    def extract(b):
        for j in range(K // 16):
            DST[b][pl.ds(j * 16, 16)] = EIV[b][1, pl.ds(j * 16, 16)]

    def start_gather(b):
        pltpu.async_copy(x_hbm.at[EIV[b].at[0]], ROWS[b], SG[b])

    def wait_gather(b):
        pltpu.make_async_copy(x_hbm.at[EIV[b].at[0]], ROWS[b], SG[b]).wait()

    def start_ea(b, base):
        pltpu.async_copy(ea_hbm.at[:, pl.ds(base, K)], EAV[b], SG[b])

    def wait_ea(b):
        pltpu.make_async_copy(ea_hbm.at[:, pl.ds(0, K)], EAV[b], SG[b]).wait()

    def fill(b):
        def fill_body(r, _):
            col = plsc.load_gather(EAV[b],
                                   [iota16, jnp.full((16,), r, jnp.int32)])
            ROWS[b][r, pl.ds(0, D_E)] = col
            return 0
        lax.fori_loop(0, K, fill_body, 0)

    def start_scatter(b):
        pltpu.async_copy(ROWS[b], acc.at[DST[b]], SS[b], add=True)

    def wait_scatter(b):
        pltpu.make_async_copy(ROWS[b], acc.at[DST[b]], SS[b]).wait()

    NP = NCHUNK // 2

    @pl.when(c == 0)
    def _():
        for b in range(2):
            start_idx(b, e0 + b * K)
        for b in range(2):
            wait_idx(b)
            extract(b)
            start_gather(b)

        def pair_body(t, _):
            for b in range(2):
                wait_gather(b)
                start_scatter(b)
                start_idx(b, e0 + (2 * t + 2 + b) * K)
            for b in range(2):
                wait_scatter(b)
                wait_idx(b)
                extract(b)
                start_gather(b)
            return 0
        lax.fori_loop(0, NP - 1, pair_body, 0)

        for b in range(2):
            wait_gather(b)
            start_scatter(b)
        for b in range(2):
            wait_scatter(b)

        @pl.when(s < (E - ETAIL0) // K)
        def _():
            start_idx(0, ETAIL0 + s * K)
            wait_idx(0)
            extract(0)
            start_gather(0)
            wait_gather(0)
            start_scatter(0)
            wait_scatter(0)

    @pl.when(c == 1)
    def _():
        for b in range(2):
            start_idx(b, e0 + b * K)
            start_ea(b, e0 + b * K)
        for b in range(2):
            wait_idx(b)
            extract(b)
            wait_ea(b)
            fill(b)

        def pair_body(t, _):
            for b in range(2):
                start_scatter(b)
                start_idx(b, e0 + (2 * t + 2 + b) * K)
                start_ea(b, e0 + (2 * t + 2 + b) * K)
            for b in range(2):
                wait_scatter(b)
                wait_idx(b)
                extract(b)
                wait_ea(b)
                fill(b)
            return 0
        lax.fori_loop(0, NP - 1, pair_body, 0)

        for b in range(2):
            start_scatter(b)
        for b in range(2):
            wait_scatter(b)

        @pl.when(s < (E - ETAIL0) // K)
        def _():
            start_idx(0, ETAIL0 + s * K)
            start_ea(0, ETAIL0 + s * K)
            wait_idx(0)
            extract(0)
            wait_ea(0)
            fill(0)
            start_scatter(0)
            wait_scatter(0)

    plsc.subcore_barrier()

    @pl.when(c == 0)
    def _():
        pltpu.sync_copy(acc.at[pl.ds(r0, RPT)], segx_out.at[pl.ds(r0, RPT)])

        @pl.when(s == NS - 1)
        def _():
            pltpu.sync_copy(acc.at[pl.ds(NS * RPT, TAIL)],
                            segx_out.at[pl.ds(NS * RPT, TAIL)])

    @pl.when(c == 1)
    def _():
        pltpu.sync_copy(acc.at[pl.ds(r0, RPT)], eac_out.at[pl.ds(r0, RPT)])

        @pl.when(s == NS - 1)
        def _():
            pltpu.sync_copy(acc.at[pl.ds(NS * RPT, TAIL)],
                            eac_out.at[pl.ds(NS * RPT, TAIL)])


NW = NC * NS
EPW = (E // (NW * K)) * K
STAIL0 = NW * EPW


@functools.partial(
    pl.kernel,
    out_type=jax.ShapeDtypeStruct((2, E), jnp.float32),
    mesh=_mesh,
    scratch_types=(
        pltpu.VMEM((4 * N,), jnp.float32),
        pltpu.VMEM((2, EPW), jnp.int32),
        pltpu.VMEM((2, EPW), jnp.float32),
        pltpu.VMEM((2, K), jnp.int32),
        pltpu.VMEM((2, K), jnp.float32),
    ),
    compiler_params=pltpu.CompilerParams(needs_layout_passes=False),
)
def _score_kernel(uv_hbm, ei_hbm, score_out, uvv, eiv, sv, eit, svt):
    c = lax.axis_index("c")
    s = lax.axis_index("s")
    w = c * NS + s
    e0 = w * EPW

    pltpu.sync_copy(uv_hbm, uvv)
    pltpu.sync_copy(ei_hbm.at[:, pl.ds(e0, EPW)], eiv)

    def make_body(idx_ref, out_ref):
        def body(i, _):
            s16 = idx_ref[0, pl.ds(i * 16, 16)] * 4
            d16 = idx_ref[1, pl.ds(i * 16, 16)] * 4
            for col in range(2):
                a = plsc.load_gather(uvv, [s16 + col])
                b = plsc.load_gather(uvv, [d16 + (col + 2)])
                out_ref[col, pl.ds(i * 16, 16)] = a + b
            return 0
        return body

    lax.fori_loop(0, EPW // 16, make_body(eiv, sv), 0)
    pltpu.sync_copy(sv, score_out.at[:, pl.ds(e0, EPW)])

    @pl.when(w < (E - STAIL0) // K)
    def _():
        t0 = STAIL0 + w * K
        pltpu.sync_copy(ei_hbm.at[:, pl.ds(t0, K)], eit)
        lax.fori_loop(0, K // 16, make_body(eit, svt), 0)
        pltpu.sync_copy(svt, score_out.at[:, pl.ds(t0, K)])


_BN = 1000


def _dense_body(x_ref, sx_ref, eac_ref,
                wx_ref, we_ref, bm_ref, wa1_ref, wa2_ref, ba_ref,
                wuv_ref, buv_ref, uv_ref):
    segx = sx_ref[...]
    sege = eac_ref[:, 0:D_E]
    cnt = eac_ref[:, D_E:D_E + 1]
    summed = (jnp.dot(segx, wx_ref[...], preferred_element_type=jnp.float32)
              + jnp.dot(sege, we_ref[...], preferred_element_type=jnp.float32)
              + cnt * bm_ref[...])
    aggr = summed / jnp.maximum(cnt, 1.0)
    h = jnp.maximum(
        jnp.dot(x_ref[...], wa1_ref[...], preferred_element_type=jnp.float32)
        + jnp.dot(aggr, wa2_ref[...], preferred_element_type=jnp.float32)
        + ba_ref[...], 0.0)
    uv_ref[...] = jnp.dot(h, wuv_ref[...],
                          preferred_element_type=jnp.float32) + buv_ref[...]


_dense = pl.pallas_call(
    _dense_body,
    grid=(N // _BN,),
    in_specs=[
        pl.BlockSpec((_BN, D_IN), lambda i: (i, 0)),
        pl.BlockSpec((_BN, D_IN), lambda i: (i, 0)),
        pl.BlockSpec((_BN, D_IN), lambda i: (i, 0)),
        pl.BlockSpec((D_IN, D_OUT), lambda i: (0, 0)),
        pl.BlockSpec((D_E, D_OUT), lambda i: (0, 0)),
        pl.BlockSpec((1, D_OUT), lambda i: (0, 0)),
        pl.BlockSpec((D_IN, D_OUT), lambda i: (0, 0)),
        pl.BlockSpec((D_OUT, D_OUT), lambda i: (0, 0)),
        pl.BlockSpec((1, D_OUT), lambda i: (0, 0)),
        pl.BlockSpec((D_OUT, 4), lambda i: (0, 0)),
        pl.BlockSpec((1, 4), lambda i: (0, 0)),
    ],
    out_specs=pl.BlockSpec((_BN, 4), lambda i: (i, 0)),
    out_shape=jax.ShapeDtypeStruct((N, 4), jnp.float32),
)


def kernel(x, edge_index, edge_attr, W_msg, b_msg, W_apply, b_apply,
           W_pred, b_pred):
    segx, eac = _segment_kernel(edge_index, x, edge_attr.T)

    wx = W_msg[:, :D_IN].T
    we = W_msg[:, D_IN:].T
    wa1 = W_apply[:, :D_IN].T
    wa2 = W_apply[:, D_IN:].T
    wuv = jnp.concatenate([W_pred[:, :D_OUT].T, W_pred[:, D_OUT:].T], axis=1)
    buv = jnp.concatenate([b_pred, jnp.zeros((2,), jnp.float32)])[None, :]

    uv = _dense(x, segx, eac,
                wx, we, b_msg[None, :], wa1, wa2, b_apply[None, :],
                wuv, buv)

    return _score_kernel(uv.reshape(-1), edge_index).T


# speedup vs baseline: 11.3739x; 1.1832x over previous
import functools

import jax
import jax.numpy as jnp
from jax import lax
from jax.experimental import pallas as pl
from jax.experimental.pallas import tpu as pltpu
from jax.experimental.pallas import tpu_sc as plsc

N = 10000
E = 320000
D_IN = 128
D_E = 16
D_OUT = 128

NC = 2
NS = 16
K = 128
NCHUNK = 156
ETAIL0 = NS * NCHUNK * K
RPT = 624
TAIL = N - NS * RPT
ZR = 104

_mesh = plsc.VectorSubcoreMesh(core_axis_name="c", subcore_axis_name="s")


@functools.partial(
    pl.kernel,
    out_type=(
        jax.ShapeDtypeStruct((N, D_IN), jnp.float32),
        jax.ShapeDtypeStruct((N, D_IN), jnp.float32),
    ),
    mesh=_mesh,
    scratch_types=(
        pltpu.VMEM((2, K), jnp.int32),
        pltpu.VMEM((2, K), jnp.int32),
        pltpu.VMEM((2, K), jnp.int32),
        pltpu.VMEM((K,), jnp.int32),
        pltpu.VMEM((K,), jnp.int32),
        pltpu.VMEM((K,), jnp.int32),
        pltpu.VMEM((K, D_IN), jnp.float32),
        pltpu.VMEM((K, D_IN), jnp.float32),
        pltpu.VMEM((K, D_IN), jnp.float32),
        pltpu.VMEM((D_E * D_E,), jnp.float32),
        pltpu.SemaphoreType.DMA,
        pltpu.SemaphoreType.DMA,
        pltpu.SemaphoreType.DMA,
        pltpu.SemaphoreType.DMA,
        pltpu.SemaphoreType.DMA,
        pltpu.SemaphoreType.DMA,
        pltpu.SemaphoreType.DMA,
        pltpu.SemaphoreType.DMA,
        pltpu.SemaphoreType.DMA,
        pltpu.VMEM_SHARED((N, D_IN), jnp.float32),
    ),
    compiler_params=pltpu.CompilerParams(needs_layout_passes=False),
)
def _segment_kernel(ei_hbm, x_hbm, ea_hbm,
                    segx_out, eac_out,
                    eiv0, eiv1, eiv2, dst0, dst1, dst2,
                    rows0, rows1, rows2, tmp,
                    si0, si1, si2, sg0, sg1, sg2, ss0, ss1, ss2, acc):
    c = lax.axis_index("c")
    s = lax.axis_index("s")
    EIV = (eiv0, eiv1, eiv2)
    DST = (dst0, dst1, dst2)
    ROWS = (rows0, rows1, rows2)
    SI = (si0, si1, si2)
    SG = (sg0, sg1, sg2)
    SS = (ss0, ss1, ss2)

    zf = jnp.zeros((16,), jnp.float32)
    onehot = jnp.where(lax.iota(jnp.int32, 16) == 0, 1.0, 0.0)

    def zero_body(i, _):
        for j in range(D_IN // 16):
            rows0[i, pl.ds(j * 16, 16)] = zf
            rows1[i, pl.ds(j * 16, 16)] = zf
            rows2[i, pl.ds(j * 16, 16)] = zf
        return 0
    lax.fori_loop(0, K, zero_body, 0)

    r0 = s * RPT
    for step in range(4):
        pltpu.sync_copy(rows0, acc.at[pl.ds(r0 + step * K, K)])
    pltpu.sync_copy(rows1.at[pl.ds(0, RPT - 4 * K)],
                    acc.at[pl.ds(r0 + 4 * K, RPT - 4 * K)])

    @pl.when(s == NS - 1)
    def _():
        pltpu.sync_copy(rows0.at[pl.ds(0, TAIL)],
                        acc.at[pl.ds(NS * RPT, TAIL)])

    @pl.when(c == 1)
    def _():
        def pre_body(i, _):
            rows0[i, pl.ds(D_E, 16)] = onehot
            rows1[i, pl.ds(D_E, 16)] = onehot
            rows2[i, pl.ds(D_E, 16)] = onehot
            return 0
        lax.fori_loop(0, K, pre_body, 0)

    plsc.subcore_barrier()

    e0 = s * (NCHUNK * K)
    iota16 = lax.iota(jnp.int32, 16)

    def start_idx(j, base):
        pltpu.async_copy(ei_hbm.at[:, pl.ds(base, K)], EIV[j], SI[j])

    def wait_idx(j):
        pltpu.make_async_copy(ei_hbm.at[:, pl.ds(0, K)], EIV[j], SI[j]).wait()

    def extract(j):
        for k in range(K // 16):
            DST[j][pl.ds(k * 16, 16)] = EIV[j][1, pl.ds(k * 16, 16)]

    def start_gather(j):
        pltpu.async_copy(x_hbm.at[EIV[j].at[0]], ROWS[j], SG[j])

    def wait_gather(j):
        pltpu.make_async_copy(x_hbm.at[EIV[j].at[0]], ROWS[j], SG[j]).wait()

    def start_ea(j, base):
        pltpu.async_copy(ea_hbm.at[:, pl.ds(base, K)],
                         ROWS[j].at[pl.ds(0, D_E)], SG[j])

    def wait_ea(j):
        pltpu.make_async_copy(ea_hbm.at[:, pl.ds(0, K)],
                              ROWS[j].at[pl.ds(0, D_E)], SG[j]).wait()

    def start_scatter(j):
        pltpu.async_copy(ROWS[j], acc.at[DST[j]], SS[j], add=True)

    def wait_scatter(j):
        pltpu.make_async_copy(ROWS[j], acc.at[DST[j]], SS[j]).wait()

    UNROLL = 8

    def fill(j):
        for i in range(D_E):
            tmp[pl.ds(i * D_E, D_E)] = ROWS[j][i, pl.ds(0, D_E)]

        def fill_body(g, _):
            for u in range(UNROLL):
                r = g * UNROLL + u + D_E
                col = plsc.load_gather(
                    ROWS[j], [iota16, jnp.full((16,), r, jnp.int32)])
                ROWS[j][r, pl.ds(0, D_E)] = col
            return 0
        lax.fori_loop(0, (K - D_E) // UNROLL, fill_body, 0)

        for r in range(D_E):
            col = plsc.load_gather(tmp, [iota16 * D_E + r])
            ROWS[j][r, pl.ds(0, D_E)] = col
            ROWS[j][r, pl.ds(D_E, 16)] = onehot
            for q in range(2, D_IN // 16):
                ROWS[j][r, pl.ds(q * 16, 16)] = zf

    NT = NCHUNK // 3 - 1

    @pl.when(c == 0)
    def _():
        for j in range(3):
            start_idx(j, e0 + j * K)
        for j in range(3):
            wait_idx(j)
            extract(j)
            start_gather(j)

        def round_body(t, _):
            for j in range(3):
                wait_gather(j)
                start_scatter(j)
                start_idx(j, e0 + (3 * t + 3 + j) * K)
            for j in range(3):
                wait_scatter(j)
                wait_idx(j)
                extract(j)
                start_gather(j)
            return 0
        lax.fori_loop(0, NT, round_body, 0)

        for j in range(3):
            wait_gather(j)
            start_scatter(j)
        for j in range(3):
            wait_scatter(j)

        @pl.when(s < (E - ETAIL0) // K)
        def _():
            start_idx(0, ETAIL0 + s * K)
            wait_idx(0)
            extract(0)
            start_gather(0)
            wait_gather(0)
            start_scatter(0)
            wait_scatter(0)

    @pl.when(c == 1)
    def _():
        for j in range(3):
            start_idx(j, e0 + j * K)
        for j in range(3):
            wait_idx(j)
            extract(j)
            start_ea(j, e0 + j * K)

        def round_body(t, _):
            for j in range(3):
                wait_ea(j)
                fill(j)
                start_scatter(j)
                start_idx(j, e0 + (3 * t + 3 + j) * K)
            for j in range(3):
                wait_scatter(j)
                wait_idx(j)
                extract(j)
                start_ea(j, e0 + (3 * t + 3 + j) * K)
            return 0
        lax.fori_loop(0, NT, round_body, 0)

        for j in range(3):
            wait_ea(j)
            fill(j)
            start_scatter(j)
        for j in range(3):
            wait_scatter(j)

        @pl.when(s < (E - ETAIL0) // K)
        def _():
            start_idx(0, ETAIL0 + s * K)
            wait_idx(0)
            extract(0)
            start_ea(0, ETAIL0 + s * K)
            wait_ea(0)
            fill(0)
            start_scatter(0)
            wait_scatter(0)

    plsc.subcore_barrier()

    @pl.when(c == 0)
    def _():
        pltpu.sync_copy(acc.at[pl.ds(r0, RPT)], segx_out.at[pl.ds(r0, RPT)])

        @pl.when(s == NS - 1)
        def _():
            pltpu.sync_copy(acc.at[pl.ds(NS * RPT, TAIL)],
                            segx_out.at[pl.ds(NS * RPT, TAIL)])

    @pl.when(c == 1)
    def _():
        pltpu.sync_copy(acc.at[pl.ds(r0, RPT)], eac_out.at[pl.ds(r0, RPT)])

        @pl.when(s == NS - 1)
        def _():
            pltpu.sync_copy(acc.at[pl.ds(NS * RPT, TAIL)],
                            eac_out.at[pl.ds(NS * RPT, TAIL)])


NW = NC * NS
EPW = (E // (NW * K)) * K
STAIL0 = NW * EPW


@functools.partial(
    pl.kernel,
    out_type=jax.ShapeDtypeStruct((2, E), jnp.float32),
    mesh=_mesh,
    scratch_types=(
        pltpu.VMEM((4 * N,), jnp.float32),
        pltpu.VMEM((2, EPW), jnp.int32),
        pltpu.VMEM((2, EPW), jnp.float32),
        pltpu.VMEM((2, K), jnp.int32),
        pltpu.VMEM((2, K), jnp.float32),
    ),
    compiler_params=pltpu.CompilerParams(needs_layout_passes=False),
)
def _score_kernel(uv_hbm, ei_hbm, score_out, uvv, eiv, sv, eit, svt):
    c = lax.axis_index("c")
    s = lax.axis_index("s")
    w = c * NS + s
    e0 = w * EPW

    pltpu.sync_copy(uv_hbm, uvv)
    pltpu.sync_copy(ei_hbm.at[:, pl.ds(e0, EPW)], eiv)

    def make_body(idx_ref, out_ref):
        def body(i, _):
            s16 = idx_ref[0, pl.ds(i * 16, 16)] * 4
            d16 = idx_ref[1, pl.ds(i * 16, 16)] * 4
            for col in range(2):
                a = plsc.load_gather(uvv, [s16 + col])
                b = plsc.load_gather(uvv, [d16 + (col + 2)])
                out_ref[col, pl.ds(i * 16, 16)] = a + b
            return 0
        return body

    lax.fori_loop(0, EPW // 16, make_body(eiv, sv), 0)
    pltpu.sync_copy(sv, score_out.at[:, pl.ds(e0, EPW)])

    @pl.when(w < (E - STAIL0) // K)
    def _():
        t0 = STAIL0 + w * K
        pltpu.sync_copy(ei_hbm.at[:, pl.ds(t0, K)], eit)
        lax.fori_loop(0, K // 16, make_body(eit, svt), 0)
        pltpu.sync_copy(svt, score_out.at[:, pl.ds(t0, K)])


_BN = 1000


def _dense_body(x_ref, sx_ref, eac_ref,
                wx_ref, we_ref, bm_ref, wa1_ref, wa2_ref, ba_ref,
                wuv_ref, buv_ref, uv_ref):
    segx = sx_ref[...]
    sege = eac_ref[:, 0:D_E]
    cnt = eac_ref[:, D_E:D_E + 1]
    summed = (jnp.dot(segx, wx_ref[...], preferred_element_type=jnp.float32)
              + jnp.dot(sege, we_ref[...], preferred_element_type=jnp.float32)
              + cnt * bm_ref[...])
    aggr = summed / jnp.maximum(cnt, 1.0)
    h = jnp.maximum(
        jnp.dot(x_ref[...], wa1_ref[...], preferred_element_type=jnp.float32)
        + jnp.dot(aggr, wa2_ref[...], preferred_element_type=jnp.float32)
        + ba_ref[...], 0.0)
    uv_ref[...] = jnp.dot(h, wuv_ref[...],
                          preferred_element_type=jnp.float32) + buv_ref[...]


_dense = pl.pallas_call(
    _dense_body,
    grid=(N // _BN,),
    in_specs=[
        pl.BlockSpec((_BN, D_IN), lambda i: (i, 0)),
        pl.BlockSpec((_BN, D_IN), lambda i: (i, 0)),
        pl.BlockSpec((_BN, D_IN), lambda i: (i, 0)),
        pl.BlockSpec((D_IN, D_OUT), lambda i: (0, 0)),
        pl.BlockSpec((D_E, D_OUT), lambda i: (0, 0)),
        pl.BlockSpec((1, D_OUT), lambda i: (0, 0)),
        pl.BlockSpec((D_IN, D_OUT), lambda i: (0, 0)),
        pl.BlockSpec((D_OUT, D_OUT), lambda i: (0, 0)),
        pl.BlockSpec((1, D_OUT), lambda i: (0, 0)),
        pl.BlockSpec((D_OUT, 4), lambda i: (0, 0)),
        pl.BlockSpec((1, 4), lambda i: (0, 0)),
    ],
    out_specs=pl.BlockSpec((_BN, 4), lambda i: (i, 0)),
    out_shape=jax.ShapeDtypeStruct((N, 4), jnp.float32),
)


def kernel(x, edge_index, edge_attr, W_msg, b_msg, W_apply, b_apply,
           W_pred, b_pred):
    segx, eac = _segment_kernel(edge_index, x, edge_attr.T)

    wx = W_msg[:, :D_IN].T
    we = W_msg[:, D_IN:].T
    wa1 = W_apply[:, :D_IN].T
    wa2 = W_apply[:, D_IN:].T
    wuv = jnp.concatenate([W_pred[:, :D_OUT].T, W_pred[:, D_OUT:].T], axis=1)
    buv = jnp.concatenate([b_pred, jnp.zeros((2,), jnp.float32)])[None, :]

    uv = _dense(x, segx, eac,
                wx, we, b_msg[None, :], wa1, wa2, b_apply[None, :],
                wuv, buv)

    return _score_kernel(uv.reshape(-1), edge_index).T


# speedup vs baseline: 11.7046x; 1.0291x over previous
import functools

import jax
import jax.numpy as jnp
from jax import lax
from jax.experimental import pallas as pl
from jax.experimental.pallas import tpu as pltpu
from jax.experimental.pallas import tpu_sc as plsc

N = 10000
E = 320000
D_IN = 128
D_E = 16
D_OUT = 128

NC = 2
NS = 16
K = 128
NCHUNK = 156
ETAIL0 = NS * NCHUNK * K
RPT = 624
TAIL = N - NS * RPT
ZR = 104

_mesh = plsc.VectorSubcoreMesh(core_axis_name="c", subcore_axis_name="s")


@functools.partial(
    pl.kernel,
    out_type=(
        jax.ShapeDtypeStruct((N, D_IN), jnp.float32),
        jax.ShapeDtypeStruct((N, D_IN), jnp.float32),
    ),
    mesh=_mesh,
    scratch_types=(
        pltpu.VMEM((2, K), jnp.int32),
        pltpu.VMEM((2, K), jnp.int32),
        pltpu.VMEM((2, K), jnp.int32),
        pltpu.VMEM((K,), jnp.int32),
        pltpu.VMEM((K,), jnp.int32),
        pltpu.VMEM((K,), jnp.int32),
        pltpu.VMEM((K, D_IN), jnp.float32),
        pltpu.VMEM((K, D_IN), jnp.float32),
        pltpu.VMEM((K, D_IN), jnp.float32),
        pltpu.VMEM((D_E * D_E,), jnp.float32),
        pltpu.SemaphoreType.DMA,
        pltpu.SemaphoreType.DMA,
        pltpu.SemaphoreType.DMA,
        pltpu.SemaphoreType.DMA,
        pltpu.SemaphoreType.DMA,
        pltpu.SemaphoreType.DMA,
        pltpu.SemaphoreType.DMA,
        pltpu.SemaphoreType.DMA,
        pltpu.SemaphoreType.DMA,
        pltpu.VMEM_SHARED((N, D_IN), jnp.float32),
    ),
    compiler_params=pltpu.CompilerParams(needs_layout_passes=False),
)
def _segment_kernel(ei_hbm, x_hbm, ea_hbm,
                    segx_out, eac_out,
                    eiv0, eiv1, eiv2, dst0, dst1, dst2,
                    rows0, rows1, rows2, tmp,
                    si0, si1, si2, sg0, sg1, sg2, ss0, ss1, ss2, acc):
    c = lax.axis_index("c")
    s = lax.axis_index("s")
    EIV = (eiv0, eiv1, eiv2)
    DST = (dst0, dst1, dst2)
    ROWS = (rows0, rows1, rows2)
    SI = (si0, si1, si2)
    SG = (sg0, sg1, sg2)
    SS = (ss0, ss1, ss2)

    zf = jnp.zeros((16,), jnp.float32)
    onehot = jnp.where(lax.iota(jnp.int32, 16) == 0, 1.0, 0.0)

    def zero_body(i, _):
        for j in range(D_IN // 16):
            rows0[i, pl.ds(j * 16, 16)] = zf
            rows1[i, pl.ds(j * 16, 16)] = zf
            rows2[i, pl.ds(j * 16, 16)] = zf
        return 0
    lax.fori_loop(0, K, zero_body, 0)

    r0 = s * RPT
    for step in range(4):
        pltpu.sync_copy(rows0, acc.at[pl.ds(r0 + step * K, K)])
    pltpu.sync_copy(rows1.at[pl.ds(0, RPT - 4 * K)],
                    acc.at[pl.ds(r0 + 4 * K, RPT - 4 * K)])

    @pl.when(s == NS - 1)
    def _():
        pltpu.sync_copy(rows0.at[pl.ds(0, TAIL)],
                        acc.at[pl.ds(NS * RPT, TAIL)])

    @pl.when(c == 1)
    def _():
        def pre_body(i, _):
            rows0[i, pl.ds(D_E, 16)] = onehot
            rows1[i, pl.ds(D_E, 16)] = onehot
            rows2[i, pl.ds(D_E, 16)] = onehot
            return 0
        lax.fori_loop(0, K, pre_body, 0)

    plsc.subcore_barrier()

    e0 = s * (NCHUNK * K)
    iota16 = lax.iota(jnp.int32, 16)

    def start_idx(j, base):
        pltpu.async_copy(ei_hbm.at[:, pl.ds(base, K)], EIV[j], SI[j])

    def wait_idx(j):
        pltpu.make_async_copy(ei_hbm.at[:, pl.ds(0, K)], EIV[j], SI[j]).wait()

    def extract(j):
        for k in range(K // 16):
            DST[j][pl.ds(k * 16, 16)] = EIV[j][1, pl.ds(k * 16, 16)]

    def start_gather(j):
        pltpu.async_copy(x_hbm.at[EIV[j].at[0]], ROWS[j], SG[j])

    def wait_gather(j):
        pltpu.make_async_copy(x_hbm.at[EIV[j].at[0]], ROWS[j], SG[j]).wait()

    def start_ea(j, base):
        pltpu.async_copy(ea_hbm.at[:, pl.ds(base, K)],
                         ROWS[j].at[pl.ds(0, D_E)], SG[j])

    def wait_ea(j):
        pltpu.make_async_copy(ea_hbm.at[:, pl.ds(0, K)],
                              ROWS[j].at[pl.ds(0, D_E)], SG[j]).wait()

    def start_scatter(j):
        pltpu.async_copy(ROWS[j], acc.at[DST[j]], SS[j], add=True)

    def wait_scatter(j):
        pltpu.make_async_copy(ROWS[j], acc.at[DST[j]], SS[j]).wait()

    UNROLL = 8

    def fill(j):
        for i in range(D_E):
            tmp[pl.ds(i * D_E, D_E)] = ROWS[j][i, pl.ds(0, D_E)]

        def fill_body(g, _):
            for u in range(UNROLL):
                r = g * UNROLL + u + D_E
                col = plsc.load_gather(
                    ROWS[j], [iota16, jnp.full((16,), r, jnp.int32)])
                ROWS[j][r, pl.ds(0, D_E)] = col
            return 0
        lax.fori_loop(0, (K - D_E) // UNROLL, fill_body, 0)

        for r in range(D_E):
            col = plsc.load_gather(tmp, [iota16 * D_E + r])
            ROWS[j][r, pl.ds(0, D_E)] = col
            ROWS[j][r, pl.ds(D_E, 16)] = onehot
            for q in range(2, D_IN // 16):
                ROWS[j][r, pl.ds(q * 16, 16)] = zf

    NT = NCHUNK // 3 - 1

    @pl.when(c == 0)
    def _():
        for j in range(3):
            start_idx(j, e0 + j * K)
        for j in range(3):
            wait_idx(j)
            extract(j)
            start_gather(j)

        def round_body(t, _):
            for j in range(3):
                wait_gather(j)
                start_scatter(j)
                start_idx(j, e0 + (3 * t + 3 + j) * K)
            for j in range(3):
                wait_scatter(j)
                wait_idx(j)
                extract(j)
                start_gather(j)
            return 0
        lax.fori_loop(0, NT, round_body, 0)

        for j in range(3):
            wait_gather(j)
            start_scatter(j)
        for j in range(3):
            wait_scatter(j)

        @pl.when(s < (E - ETAIL0) // K)
        def _():
            start_idx(0, ETAIL0 + s * K)
            wait_idx(0)
            extract(0)
            start_gather(0)
            wait_gather(0)
            start_scatter(0)
            wait_scatter(0)

    @pl.when(c == 1)
    def _():
        for j in range(3):
            start_idx(j, e0 + j * K)
        for j in range(3):
            wait_idx(j)
            extract(j)
            start_ea(j, e0 + j * K)

        def round_body(t, _):
            for j in range(3):
                wait_ea(j)
                fill(j)
                start_scatter(j)
                start_idx(j, e0 + (3 * t + 3 + j) * K)
            for j in range(3):
                wait_scatter(j)
                wait_idx(j)
                extract(j)
                start_ea(j, e0 + (3 * t + 3 + j) * K)
            return 0
        lax.fori_loop(0, NT, round_body, 0)

        for j in range(3):
            wait_ea(j)
            fill(j)
            start_scatter(j)
        for j in range(3):
            wait_scatter(j)

        @pl.when(s < (E - ETAIL0) // K)
        def _():
            start_idx(0, ETAIL0 + s * K)
            wait_idx(0)
            extract(0)
            start_ea(0, ETAIL0 + s * K)
            wait_ea(0)
            fill(0)
            start_scatter(0)
            wait_scatter(0)

    plsc.subcore_barrier()

    @pl.when(c == 0)
    def _():
        pltpu.sync_copy(acc.at[pl.ds(r0, RPT)], segx_out.at[pl.ds(r0, RPT)])

        @pl.when(s == NS - 1)
        def _():
            pltpu.sync_copy(acc.at[pl.ds(NS * RPT, TAIL)],
                            segx_out.at[pl.ds(NS * RPT, TAIL)])

    @pl.when(c == 1)
    def _():
        pltpu.sync_copy(acc.at[pl.ds(r0, RPT)], eac_out.at[pl.ds(r0, RPT)])

        @pl.when(s == NS - 1)
        def _():
            pltpu.sync_copy(acc.at[pl.ds(NS * RPT, TAIL)],
                            eac_out.at[pl.ds(NS * RPT, TAIL)])


NW = NC * NS
EPW = (E // (NW * K)) * K
STAIL0 = NW * EPW


@functools.partial(
    pl.kernel,
    out_type=jax.ShapeDtypeStruct((2, E), jnp.float32),
    mesh=_mesh,
    scratch_types=(
        pltpu.VMEM((4, N), jnp.float32),
        pltpu.VMEM((2, EPW), jnp.int32),
        pltpu.VMEM((2, EPW), jnp.float32),
        pltpu.VMEM((2, K), jnp.int32),
        pltpu.VMEM((2, K), jnp.float32),
    ),
    compiler_params=pltpu.CompilerParams(needs_layout_passes=False),
)
def _score_kernel(uv_hbm, ei_hbm, score_out, uvv, eiv, sv, eit, svt):
    c = lax.axis_index("c")
    s = lax.axis_index("s")
    w = c * NS + s
    e0 = w * EPW

    pltpu.sync_copy(uv_hbm, uvv)
    pltpu.sync_copy(ei_hbm.at[:, pl.ds(e0, EPW)], eiv)

    def make_body(idx_ref, out_ref):
        def body(i, _):
            s16 = idx_ref[0, pl.ds(i * 16, 16)]
            d16 = idx_ref[1, pl.ds(i * 16, 16)]
            for col in range(2):
                a = plsc.load_gather(
                    uvv, [jnp.full((16,), col, jnp.int32), s16])
                b = plsc.load_gather(
                    uvv, [jnp.full((16,), col + 2, jnp.int32), d16])
                out_ref[col, pl.ds(i * 16, 16)] = a + b
            return 0
        return body

    lax.fori_loop(0, EPW // 16, make_body(eiv, sv), 0)
    pltpu.sync_copy(sv, score_out.at[:, pl.ds(e0, EPW)])

    @pl.when(w < (E - STAIL0) // K)
    def _():
        t0 = STAIL0 + w * K
        pltpu.sync_copy(ei_hbm.at[:, pl.ds(t0, K)], eit)
        lax.fori_loop(0, K // 16, make_body(eit, svt), 0)
        pltpu.sync_copy(svt, score_out.at[:, pl.ds(t0, K)])


def _dense_body(x_ref, sx_ref, eac_ref,
                wx_ref, we_ref, bm_ref, wa1_ref, wa2_ref, ba_ref,
                wuv_ref, buv_ref, uv_ref):
    segx = sx_ref[...]
    sege = eac_ref[:, 0:D_E]
    cnt = eac_ref[:, D_E:D_E + 1]
    summed = (jnp.dot(segx, wx_ref[...], preferred_element_type=jnp.float32)
              + jnp.dot(sege, we_ref[...], preferred_element_type=jnp.float32)
              + cnt * bm_ref[...])
    aggr = summed / jnp.maximum(cnt, 1.0)
    h = jnp.maximum(
        jnp.dot(x_ref[...], wa1_ref[...], preferred_element_type=jnp.float32)
        + jnp.dot(aggr, wa2_ref[...], preferred_element_type=jnp.float32)
        + ba_ref[...], 0.0)
    uv_ref[...] = lax.dot_general(
        wuv_ref[...], h, (((1,), (1,)), ((), ())),
        preferred_element_type=jnp.float32) + buv_ref[...]


_dense = pl.pallas_call(
    _dense_body,
    out_shape=jax.ShapeDtypeStruct((4, N), jnp.float32),
)


def kernel(x, edge_index, edge_attr, W_msg, b_msg, W_apply, b_apply,
           W_pred, b_pred):
    segx, eac = _segment_kernel(edge_index, x, edge_attr.T)

    wx = W_msg[:, :D_IN].T
    we = W_msg[:, D_IN:].T
    wa1 = W_apply[:, :D_IN].T
    wa2 = W_apply[:, D_IN:].T
    wuv = jnp.concatenate([W_pred[:, :D_OUT], W_pred[:, D_OUT:]], axis=0)
    buv = jnp.concatenate([b_pred, jnp.zeros((2,), jnp.float32)])[:, None]

    uv = _dense(x, segx, eac,
                wx, we, b_msg[None, :], wa1, wa2, b_apply[None, :],
                wuv, buv)

    return _score_kernel(uv, edge_index).T


# speedup vs baseline: 13.3391x; 1.1396x over previous
import functools

import jax
import jax.numpy as jnp
from jax import lax
from jax.experimental import pallas as pl
from jax.experimental.pallas import tpu as pltpu
from jax.experimental.pallas import tpu_sc as plsc

N = 10000
E = 320000
D_IN = 128
D_E = 16
D_OUT = 128

NC = 2
NS = 16
K = 128
NCHUNK = 156
ETAIL0 = NS * NCHUNK * K
RPT = 624
TAIL = N - NS * RPT
ZR = 104

_mesh = plsc.VectorSubcoreMesh(core_axis_name="c", subcore_axis_name="s")


@functools.partial(
    pl.kernel,
    out_type=(
        jax.ShapeDtypeStruct((N, D_IN), jnp.float32),
        jax.ShapeDtypeStruct((N, D_IN), jnp.float32),
    ),
    mesh=_mesh,
    scratch_types=(
        pltpu.VMEM((2, K), jnp.int32),
        pltpu.VMEM((2, K), jnp.int32),
        pltpu.VMEM((2, K), jnp.int32),
        pltpu.VMEM((K,), jnp.int32),
        pltpu.VMEM((K,), jnp.int32),
        pltpu.VMEM((K,), jnp.int32),
        pltpu.VMEM((K, D_IN), jnp.float32),
        pltpu.VMEM((K, D_IN), jnp.float32),
        pltpu.VMEM((K, D_IN), jnp.float32),
        pltpu.VMEM((D_E * D_E,), jnp.float32),
        pltpu.SemaphoreType.DMA,
        pltpu.SemaphoreType.DMA,
        pltpu.SemaphoreType.DMA,
        pltpu.SemaphoreType.DMA,
        pltpu.SemaphoreType.DMA,
        pltpu.SemaphoreType.DMA,
        pltpu.SemaphoreType.DMA,
        pltpu.SemaphoreType.DMA,
        pltpu.SemaphoreType.DMA,
        pltpu.VMEM_SHARED((N, D_IN), jnp.float32),
    ),
    compiler_params=pltpu.CompilerParams(needs_layout_passes=False),
)
def _segment_kernel(ei_hbm, x_hbm, ea_hbm,
                    segx_out, eac_out,
                    eiv0, eiv1, eiv2, dst0, dst1, dst2,
                    rows0, rows1, rows2, tmp,
                    si0, si1, si2, sg0, sg1, sg2, ss0, ss1, ss2, acc):
    c = lax.axis_index("c")
    s = lax.axis_index("s")
    EIV = (eiv0, eiv1, eiv2)
    DST = (dst0, dst1, dst2)
    ROWS = (rows0, rows1, rows2)
    SI = (si0, si1, si2)
    SG = (sg0, sg1, sg2)
    SS = (ss0, ss1, ss2)

    zf = jnp.zeros((16,), jnp.float32)
    onehot = jnp.where(lax.iota(jnp.int32, 16) == 0, 1.0, 0.0)

    def zero_body(i, _):
        for j in range(D_IN // 16):
            rows0[i, pl.ds(j * 16, 16)] = zf
            rows1[i, pl.ds(j * 16, 16)] = zf
            rows2[i, pl.ds(j * 16, 16)] = zf
        return 0
    lax.fori_loop(0, K, zero_body, 0)

    r0 = s * RPT
    for step in range(4):
        pltpu.sync_copy(rows0, acc.at[pl.ds(r0 + step * K, K)])
    pltpu.sync_copy(rows1.at[pl.ds(0, RPT - 4 * K)],
                    acc.at[pl.ds(r0 + 4 * K, RPT - 4 * K)])

    @pl.when(s == NS - 1)
    def _():
        pltpu.sync_copy(rows0.at[pl.ds(0, TAIL)],
                        acc.at[pl.ds(NS * RPT, TAIL)])

    @pl.when(c == 1)
    def _():
        def pre_body(i, _):
            rows0[i, pl.ds(D_E, 16)] = onehot
            rows1[i, pl.ds(D_E, 16)] = onehot
            rows2[i, pl.ds(D_E, 16)] = onehot
            return 0
        lax.fori_loop(0, K, pre_body, 0)

    plsc.subcore_barrier()

    e0 = s * (NCHUNK * K)
    iota16 = lax.iota(jnp.int32, 16)

    def start_idx(j, base):
        pltpu.async_copy(ei_hbm.at[:, pl.ds(base, K)], EIV[j], SI[j])

    def wait_idx(j):
        pltpu.make_async_copy(ei_hbm.at[:, pl.ds(0, K)], EIV[j], SI[j]).wait()

    def extract(j):
        for k in range(K // 16):
            DST[j][pl.ds(k * 16, 16)] = EIV[j][1, pl.ds(k * 16, 16)]

    def start_gather(j):
        pltpu.async_copy(x_hbm.at[EIV[j].at[0]], ROWS[j], SG[j])

    def wait_gather(j):
        pltpu.make_async_copy(x_hbm.at[EIV[j].at[0]], ROWS[j], SG[j]).wait()

    def start_ea(j, base):
        pltpu.async_copy(ea_hbm.at[:, pl.ds(base, K)],
                         ROWS[j].at[pl.ds(0, D_E)], SG[j])

    def wait_ea(j):
        pltpu.make_async_copy(ea_hbm.at[:, pl.ds(0, K)],
                              ROWS[j].at[pl.ds(0, D_E)], SG[j]).wait()

    def start_scatter(j):
        pltpu.async_copy(ROWS[j], acc.at[DST[j]], SS[j], add=True)

    def wait_scatter(j):
        pltpu.make_async_copy(ROWS[j], acc.at[DST[j]], SS[j]).wait()

    UNROLL = 8

    def fill(j):
        for i in range(D_E):
            tmp[pl.ds(i * D_E, D_E)] = ROWS[j][i, pl.ds(0, D_E)]

        def fill_body(g, _):
            r0 = g * 16 + D_E
            for f in range(D_E):
                vec = ROWS[j][f, pl.ds(r0, 16)]
                plsc.store_scatter(ROWS[j], [iota16 + r0,
                                             jnp.full((16,), f, jnp.int32)],
                                   vec)
            return 0
        lax.fori_loop(0, (K - D_E) // 16, fill_body, 0)

        for r in range(D_E):
            col = plsc.load_gather(tmp, [iota16 * D_E + r])
            ROWS[j][r, pl.ds(0, D_E)] = col
            ROWS[j][r, pl.ds(D_E, 16)] = onehot
            for q in range(2, D_IN // 16):
                ROWS[j][r, pl.ds(q * 16, 16)] = zf

    NT = NCHUNK // 3 - 1

    @pl.when(c == 0)
    def _():
        for j in range(3):
            start_idx(j, e0 + j * K)
        for j in range(3):
            wait_idx(j)
            extract(j)
            start_gather(j)

        def round_body(t, _):
            for j in range(3):
                wait_gather(j)
                start_scatter(j)
                start_idx(j, e0 + (3 * t + 3 + j) * K)
            for j in range(3):
                wait_scatter(j)
                wait_idx(j)
                extract(j)
                start_gather(j)
            return 0
        lax.fori_loop(0, NT, round_body, 0)

        for j in range(3):
            wait_gather(j)
            start_scatter(j)
        for j in range(3):
            wait_scatter(j)

        @pl.when(s < (E - ETAIL0) // K)
        def _():
            start_idx(0, ETAIL0 + s * K)
            wait_idx(0)
            extract(0)
            start_gather(0)
            wait_gather(0)
            start_scatter(0)
            wait_scatter(0)

    @pl.when(c == 1)
    def _():
        for j in range(3):
            start_idx(j, e0 + j * K)
        for j in range(3):
            wait_idx(j)
            extract(j)
            start_ea(j, e0 + j * K)

        def round_body(t, _):
            for j in range(3):
                wait_ea(j)
                fill(j)
                start_scatter(j)
                start_idx(j, e0 + (3 * t + 3 + j) * K)
            for j in range(3):
                wait_scatter(j)
                wait_idx(j)
                extract(j)
                start_ea(j, e0 + (3 * t + 3 + j) * K)
            return 0
        lax.fori_loop(0, NT, round_body, 0)

        for j in range(3):
            wait_ea(j)
            fill(j)
            start_scatter(j)
        for j in range(3):
            wait_scatter(j)

        @pl.when(s < (E - ETAIL0) // K)
        def _():
            start_idx(0, ETAIL0 + s * K)
            wait_idx(0)
            extract(0)
            start_ea(0, ETAIL0 + s * K)
            wait_ea(0)
            fill(0)
            start_scatter(0)
            wait_scatter(0)

    plsc.subcore_barrier()

    @pl.when(c == 0)
    def _():
        pltpu.sync_copy(acc.at[pl.ds(r0, RPT)], segx_out.at[pl.ds(r0, RPT)])

        @pl.when(s == NS - 1)
        def _():
            pltpu.sync_copy(acc.at[pl.ds(NS * RPT, TAIL)],
                            segx_out.at[pl.ds(NS * RPT, TAIL)])

    @pl.when(c == 1)
    def _():
        pltpu.sync_copy(acc.at[pl.ds(r0, RPT)], eac_out.at[pl.ds(r0, RPT)])

        @pl.when(s == NS - 1)
        def _():
            pltpu.sync_copy(acc.at[pl.ds(NS * RPT, TAIL)],
                            eac_out.at[pl.ds(NS * RPT, TAIL)])


NW = NC * NS
EPW = (E // (NW * K)) * K
STAIL0 = NW * EPW


@functools.partial(
    pl.kernel,
    out_type=jax.ShapeDtypeStruct((2, E), jnp.float32),
    mesh=_mesh,
    scratch_types=(
        pltpu.VMEM((4, N), jnp.float32),
        pltpu.VMEM((2, EPW), jnp.int32),
        pltpu.VMEM((2, EPW), jnp.float32),
        pltpu.VMEM((2, K), jnp.int32),
        pltpu.VMEM((2, K), jnp.float32),
    ),
    compiler_params=pltpu.CompilerParams(needs_layout_passes=False),
)
def _score_kernel(uv_hbm, ei_hbm, score_out, uvv, eiv, sv, eit, svt):
    c = lax.axis_index("c")
    s = lax.axis_index("s")
    w = c * NS + s
    e0 = w * EPW

    pltpu.sync_copy(uv_hbm, uvv)
    pltpu.sync_copy(ei_hbm.at[:, pl.ds(e0, EPW)], eiv)

    def make_body(idx_ref, out_ref):
        def body(i, _):
            s16 = idx_ref[0, pl.ds(i * 16, 16)]
            d16 = idx_ref[1, pl.ds(i * 16, 16)]
            for col in range(2):
                a = plsc.load_gather(
                    uvv, [jnp.full((16,), col, jnp.int32), s16])
                b = plsc.load_gather(
                    uvv, [jnp.full((16,), col + 2, jnp.int32), d16])
                out_ref[col, pl.ds(i * 16, 16)] = a + b
            return 0
        return body

    lax.fori_loop(0, EPW // 16, make_body(eiv, sv), 0)
    pltpu.sync_copy(sv, score_out.at[:, pl.ds(e0, EPW)])

    @pl.when(w < (E - STAIL0) // K)
    def _():
        t0 = STAIL0 + w * K
        pltpu.sync_copy(ei_hbm.at[:, pl.ds(t0, K)], eit)
        lax.fori_loop(0, K // 16, make_body(eit, svt), 0)
        pltpu.sync_copy(svt, score_out.at[:, pl.ds(t0, K)])


def _dense_body(x_ref, sx_ref, eac_ref,
                wx_ref, we_ref, bm_ref, wa1_ref, wa2_ref, ba_ref,
                wuv_ref, buv_ref, uv_ref):
    segx = sx_ref[...]
    sege = eac_ref[:, 0:D_E]
    cnt = eac_ref[:, D_E:D_E + 1]
    summed = (jnp.dot(segx, wx_ref[...], preferred_element_type=jnp.float32)
              + jnp.dot(sege, we_ref[...], preferred_element_type=jnp.float32)
              + cnt * bm_ref[...])
    aggr = summed / jnp.maximum(cnt, 1.0)
    h = jnp.maximum(
        jnp.dot(x_ref[...], wa1_ref[...], preferred_element_type=jnp.float32)
        + jnp.dot(aggr, wa2_ref[...], preferred_element_type=jnp.float32)
        + ba_ref[...], 0.0)
    uv_ref[...] = lax.dot_general(
        wuv_ref[...], h, (((1,), (1,)), ((), ())),
        preferred_element_type=jnp.float32) + buv_ref[...]


_dense = pl.pallas_call(
    _dense_body,
    out_shape=jax.ShapeDtypeStruct((4, N), jnp.float32),
)


def kernel(x, edge_index, edge_attr, W_msg, b_msg, W_apply, b_apply,
           W_pred, b_pred):
    segx, eac = _segment_kernel(edge_index, x, edge_attr.T)

    wx = W_msg[:, :D_IN].T
    we = W_msg[:, D_IN:].T
    wa1 = W_apply[:, :D_IN].T
    wa2 = W_apply[:, D_IN:].T
    wuv = jnp.concatenate([W_pred[:, :D_OUT], W_pred[:, D_OUT:]], axis=0)
    buv = jnp.concatenate([b_pred, jnp.zeros((2,), jnp.float32)])[:, None]

    uv = _dense(x, segx, eac,
                wx, we, b_msg[None, :], wa1, wa2, b_apply[None, :],
                wuv, buv)

    return _score_kernel(uv, edge_index).T


# speedup vs baseline: 13.3427x; 1.0003x over previous
import functools

import jax
import jax.numpy as jnp
from jax import lax
from jax.experimental import pallas as pl
from jax.experimental.pallas import tpu as pltpu
from jax.experimental.pallas import tpu_sc as plsc

N = 10000
E = 320000
D_IN = 128
D_E = 16
D_OUT = 128

NC = 2
NS = 16
K = 128
NCHUNK = 156
ETAIL0 = NS * NCHUNK * K
RPT = 624
TAIL = N - NS * RPT

_mesh = plsc.VectorSubcoreMesh(core_axis_name="c", subcore_axis_name="s")


@functools.partial(
    pl.kernel,
    out_type=(
        jax.ShapeDtypeStruct((N, D_IN), jnp.float32),
        jax.ShapeDtypeStruct((N, D_IN), jnp.float32),
    ),
    mesh=_mesh,
    scratch_types=(
        pltpu.VMEM((2, K), jnp.int32),
        pltpu.VMEM((2, K), jnp.int32),
        pltpu.VMEM((2, K), jnp.int32),
        pltpu.VMEM((K,), jnp.int32),
        pltpu.VMEM((K,), jnp.int32),
        pltpu.VMEM((K,), jnp.int32),
        pltpu.VMEM((K, D_IN), jnp.float32),
        pltpu.VMEM((K, D_IN), jnp.float32),
        pltpu.VMEM((K, D_IN), jnp.float32),
        pltpu.VMEM((D_E * D_E,), jnp.float32),
        pltpu.SemaphoreType.DMA,
        pltpu.SemaphoreType.DMA,
        pltpu.SemaphoreType.DMA,
        pltpu.SemaphoreType.DMA,
        pltpu.SemaphoreType.DMA,
        pltpu.SemaphoreType.DMA,
        pltpu.SemaphoreType.DMA,
        pltpu.SemaphoreType.DMA,
        pltpu.SemaphoreType.DMA,
        pltpu.VMEM_SHARED((N, D_IN), jnp.float32),
    ),
    compiler_params=pltpu.CompilerParams(needs_layout_passes=False),
)
def _segment_kernel(ei_hbm, x_hbm, ea_hbm,
                    segx_out, eac_out,
                    eiv0, eiv1, eiv2, dst0, dst1, dst2,
                    rows0, rows1, rows2, tmp,
                    si0, si1, si2, sg0, sg1, sg2, ss0, ss1, ss2, acc):
    c = lax.axis_index("c")
    s = lax.axis_index("s")
    EIV = (eiv0, eiv1, eiv2)
    DST = (dst0, dst1, dst2)
    ROWS = (rows0, rows1, rows2)
    SI = (si0, si1, si2)
    SG = (sg0, sg1, sg2)
    SS = (ss0, ss1, ss2)

    zf = jnp.zeros((16,), jnp.float32)
    onehot = jnp.where(lax.iota(jnp.int32, 16) == 0, 1.0, 0.0)

    def zero_body(i, _):
        for j in range(D_IN // 16):
            rows0[i, pl.ds(j * 16, 16)] = zf
            rows1[i, pl.ds(j * 16, 16)] = zf
            rows2[i, pl.ds(j * 16, 16)] = zf
        return 0
    lax.fori_loop(0, K, zero_body, 0)

    r0 = s * RPT
    for step in range(4):
        pltpu.sync_copy(rows0, acc.at[pl.ds(r0 + step * K, K)])
    pltpu.sync_copy(rows1.at[pl.ds(0, RPT - 4 * K)],
                    acc.at[pl.ds(r0 + 4 * K, RPT - 4 * K)])

    @pl.when(s == NS - 1)
    def _():
        pltpu.sync_copy(rows0.at[pl.ds(0, TAIL)],
                        acc.at[pl.ds(NS * RPT, TAIL)])

    @pl.when(c == 1)
    def _():
        def pre_body(i, _):
            rows0[i, pl.ds(D_E, 16)] = onehot
            rows1[i, pl.ds(D_E, 16)] = onehot
            rows2[i, pl.ds(D_E, 16)] = onehot
            return 0
        lax.fori_loop(0, K, pre_body, 0)

    plsc.subcore_barrier()

    e0 = s * (NCHUNK * K)
    iota16 = lax.iota(jnp.int32, 16)

    def start_idx(j, base):
        pltpu.async_copy(ei_hbm.at[:, pl.ds(base, K)], EIV[j], SI[j])

    def wait_idx(j):
        pltpu.make_async_copy(ei_hbm.at[:, pl.ds(0, K)], EIV[j], SI[j]).wait()

    def extract(j):
        for k in range(K // 16):
            DST[j][pl.ds(k * 16, 16)] = EIV[j][1, pl.ds(k * 16, 16)]

    def start_gather(j):
        pltpu.async_copy(x_hbm.at[EIV[j].at[0]], ROWS[j], SG[j])

    def wait_gather(j):
        pltpu.make_async_copy(x_hbm.at[EIV[j].at[0]], ROWS[j], SG[j]).wait()

    def start_ea(j, base):
        pltpu.async_copy(ea_hbm.at[:, pl.ds(base, K)],
                         ROWS[j].at[pl.ds(0, D_E)], SG[j])

    def wait_ea(j):
        pltpu.make_async_copy(ea_hbm.at[:, pl.ds(0, K)],
                              ROWS[j].at[pl.ds(0, D_E)], SG[j]).wait()

    def start_scatter(j):
        pltpu.async_copy(ROWS[j], acc.at[DST[j]], SS[j], add=True)

    def wait_scatter(j):
        pltpu.make_async_copy(ROWS[j], acc.at[DST[j]], SS[j]).wait()

    def fill(j):
        for i in range(D_E):
            tmp[pl.ds(i * D_E, D_E)] = ROWS[j][i, pl.ds(0, D_E)]

        def fill_body(g, _):
            r0 = g * 16 + D_E
            for f in range(D_E):
                vec = ROWS[j][f, pl.ds(r0, 16)]
                plsc.store_scatter(ROWS[j], [iota16 + r0,
                                             jnp.full((16,), f, jnp.int32)],
                                   vec)
            return 0
        lax.fori_loop(0, (K - D_E) // 16, fill_body, 0)

        for r in range(D_E):
            col = plsc.load_gather(tmp, [iota16 * D_E + r])
            ROWS[j][r, pl.ds(0, D_E)] = col
            ROWS[j][r, pl.ds(D_E, 16)] = onehot
            for q in range(2, D_IN // 16):
                ROWS[j][r, pl.ds(q * 16, 16)] = zf

    NT = NCHUNK // 3 - 1

    @pl.when(c == 0)
    def _():
        for j in range(3):
            start_idx(j, e0 + j * K)
        for j in range(3):
            wait_idx(j)
            extract(j)
            start_gather(j)

        def round_body(t, _):
            for j in range(3):
                wait_gather(j)
                start_scatter(j)
                start_idx(j, e0 + (3 * t + 3 + j) * K)
            for j in range(3):
                wait_scatter(j)
                wait_idx(j)
                extract(j)
                start_gather(j)
            return 0
        lax.fori_loop(0, NT, round_body, 0)

        for j in range(3):
            wait_gather(j)
            start_scatter(j)
        for j in range(3):
            wait_scatter(j)

        @pl.when(s < (E - ETAIL0) // K)
        def _():
            start_idx(0, ETAIL0 + s * K)
            wait_idx(0)
            extract(0)
            start_gather(0)
            wait_gather(0)
            start_scatter(0)
            wait_scatter(0)

    @pl.when(c == 1)
    def _():
        for j in range(3):
            start_idx(j, e0 + j * K)
        for j in range(3):
            wait_idx(j)
            extract(j)
            start_ea(j, e0 + j * K)

        def round_body(t, _):
            for j in range(3):
                wait_ea(j)
                fill(j)
                start_scatter(j)
                start_idx(j, e0 + (3 * t + 3 + j) * K)
            for j in range(3):
                wait_scatter(j)
                wait_idx(j)
                extract(j)
                start_ea(j, e0 + (3 * t + 3 + j) * K)
            return 0
        lax.fori_loop(0, NT, round_body, 0)

        for j in range(3):
            wait_ea(j)
            fill(j)
            start_scatter(j)
        for j in range(3):
            wait_scatter(j)

        @pl.when(s < (E - ETAIL0) // K)
        def _():
            start_idx(0, ETAIL0 + s * K)
            wait_idx(0)
            extract(0)
            start_ea(0, ETAIL0 + s * K)
            wait_ea(0)
            fill(0)
            start_scatter(0)
            wait_scatter(0)

    plsc.subcore_barrier()

    @pl.when(c == 0)
    def _():
        pltpu.sync_copy(acc.at[pl.ds(r0, RPT)], segx_out.at[pl.ds(r0, RPT)])

        @pl.when(s == NS - 1)
        def _():
            pltpu.sync_copy(acc.at[pl.ds(NS * RPT, TAIL)],
                            segx_out.at[pl.ds(NS * RPT, TAIL)])

    @pl.when(c == 1)
    def _():
        pltpu.sync_copy(acc.at[pl.ds(r0, RPT)], eac_out.at[pl.ds(r0, RPT)])

        @pl.when(s == NS - 1)
        def _():
            pltpu.sync_copy(acc.at[pl.ds(NS * RPT, TAIL)],
                            eac_out.at[pl.ds(NS * RPT, TAIL)])


NW = NC * NS
EPW = (E // (NW * K)) * K
STAIL0 = NW * EPW


@functools.partial(
    pl.kernel,
    out_type=jax.ShapeDtypeStruct((2, E), jnp.float32),
    mesh=_mesh,
    scratch_types=(
        pltpu.VMEM((4, N), jnp.float32),
        pltpu.VMEM((2, EPW), jnp.int32),
        pltpu.VMEM((2, EPW), jnp.float32),
        pltpu.VMEM((2, K), jnp.int32),
        pltpu.VMEM((2, K), jnp.float32),
    ),
    compiler_params=pltpu.CompilerParams(needs_layout_passes=False),
)
def _score_kernel(uv_hbm, ei_hbm, score_out, uvv, eiv, sv, eit, svt):
    c = lax.axis_index("c")
    s = lax.axis_index("s")
    w = c * NS + s
    e0 = w * EPW

    pltpu.sync_copy(uv_hbm, uvv)
    pltpu.sync_copy(ei_hbm.at[:, pl.ds(e0, EPW)], eiv)

    def make_body(idx_ref, out_ref):
        def body(i, _):
            s16 = idx_ref[0, pl.ds(i * 16, 16)]
            d16 = idx_ref[1, pl.ds(i * 16, 16)]
            for col in range(2):
                a = plsc.load_gather(
                    uvv, [jnp.full((16,), col, jnp.int32), s16])
                b = plsc.load_gather(
                    uvv, [jnp.full((16,), col + 2, jnp.int32), d16])
                out_ref[col, pl.ds(i * 16, 16)] = a + b
            return 0
        return body

    lax.fori_loop(0, EPW // 16, make_body(eiv, sv), 0)
    pltpu.sync_copy(sv, score_out.at[:, pl.ds(e0, EPW)])

    @pl.when(w < (E - STAIL0) // K)
    def _():
        t0 = STAIL0 + w * K
        pltpu.sync_copy(ei_hbm.at[:, pl.ds(t0, K)], eit)
        lax.fori_loop(0, K // 16, make_body(eit, svt), 0)
        pltpu.sync_copy(svt, score_out.at[:, pl.ds(t0, K)])


def _dense_body(x_ref, sx_ref, eac_ref,
                wx_ref, we_ref, bm_ref, wa1_ref, wa2_ref, ba_ref,
                wuv_ref, buv_ref, uv_ref):
    segx = sx_ref[...]
    sege = eac_ref[:, 0:D_E]
    cnt = eac_ref[:, D_E:D_E + 1]
    summed = (jnp.dot(segx, wx_ref[...], preferred_element_type=jnp.float32)
              + jnp.dot(sege, we_ref[...], preferred_element_type=jnp.float32)
              + cnt * bm_ref[...])
    aggr = summed / jnp.maximum(cnt, 1.0)
    h = jnp.maximum(
        jnp.dot(x_ref[...], wa1_ref[...], preferred_element_type=jnp.float32)
        + jnp.dot(aggr, wa2_ref[...], preferred_element_type=jnp.float32)
        + ba_ref[...], 0.0)
    uv_ref[...] = lax.dot_general(
        wuv_ref[...], h, (((1,), (1,)), ((), ())),
        preferred_element_type=jnp.float32) + buv_ref[...]


_dense = pl.pallas_call(
    _dense_body,
    out_shape=jax.ShapeDtypeStruct((4, N), jnp.float32),
)


def kernel(x, edge_index, edge_attr, W_msg, b_msg, W_apply, b_apply,
           W_pred, b_pred):
    segx, eac = _segment_kernel(edge_index, x, edge_attr.T)

    wx = W_msg[:, :D_IN].T
    we = W_msg[:, D_IN:].T
    wa1 = W_apply[:, :D_IN].T
    wa2 = W_apply[:, D_IN:].T
    wuv = jnp.concatenate([W_pred[:, :D_OUT], W_pred[:, D_OUT:]], axis=0)
    buv = jnp.concatenate([b_pred, jnp.zeros((2,), jnp.float32)])[:, None]

    uv = _dense(x, segx, eac,
                wx, we, b_msg[None, :], wa1, wa2, b_apply[None, :],
                wuv, buv)

    return _score_kernel(uv, edge_index).T
